# Initial kernel scaffold; baseline (speedup 1.0000x reference)
import jax
import jax.numpy as jnp
from jax.experimental import pallas as pl


def kernel(input_ids, token_emb, pos_emb, gamma, beta):
    raise NotImplementedError("write your pallas kernel here")


# speedup vs baseline: 2.6958x; 2.6958x over previous
import functools

import jax
import jax.numpy as jnp
from jax import lax
from jax.experimental import pallas as pl
from jax.experimental.pallas import tpu as pltpu
from jax.experimental.pallas import tpu_sc as plsc

D = 128
B = 1024
L = 200
EPS = 1e-12
LP = 208
NC = 2
NS = 16
NW = NC * NS
ROWS_PER_W = B // NW
NCHUNK = 13

_LANE = None


_DNUMS = lax.GatherDimensionNumbers(
    offset_dims=(), collapsed_slice_dims=(0,), start_index_map=(0,))


def _take(v, idx):
    return lax.gather(v, idx[:, None], _DNUMS, (1,),
                      mode=lax.GatherScatterMode.PROMISE_IN_BOUNDS)


def _butterfly_sum(v, lane):
    for sh in (1, 2, 4, 8):
        v = v + _take(v, lane ^ sh)
    return v


def _sc_body(ids_hbm, tok_hbm, pos_hbm, gamma_hbm, beta_hbm, out_hbm,
             idx_v, pidx_v, rows_v, prow_v, out_v, gam_v, bet_v,
             sem_t, sem_p):
    wid = lax.axis_index("s") * NC + lax.axis_index("c")
    pltpu.sync_copy(gamma_hbm, gam_v)
    pltpu.sync_copy(beta_hbm, bet_v)
    lane = lax.iota(jnp.int32, 16)
    gs = [gam_v[pl.ds(16 * k, 16)] for k in range(8)]
    bs = [bet_v[pl.ds(16 * k, 16)] for k in range(8)]
    fifteen = lane * 0 + 15

    def row_body(r, carry0):
        row = wid * ROWS_PER_W + r
        pltpu.sync_copy(ids_hbm.at[pl.ds(row * L, L)], idx_v.at[pl.ds(0, L)])
        tail = idx_v[pl.ds(192, 16)]
        idx_v[pl.ds(192, 16)] = jnp.where(lane < 8, tail, 0)
        carry = lane * 0
        for k in range(NCHUNK):
            ids_k = idx_v[pl.ds(16 * k, 16)]
            m = jnp.minimum(ids_k, 1)
            c = m
            for sh in (1, 2, 4, 8):
                c = c + jnp.where(lane >= sh,
                                  _take(c, jnp.maximum(lane - sh, 0)), 0)
            pidx_v[pl.ds(16 * k, 16)] = (c + carry) * m
            carry = carry + _take(c, fifteen)
        cps = []
        for j in range(2):
            sl = pl.ds(104 * j, 104)
            cps.append(pltpu.async_copy(
                tok_hbm.at[idx_v.at[sl]], rows_v.at[sl], sem_t))
            cps.append(pltpu.async_copy(
                pos_hbm.at[pidx_v.at[sl]], prow_v.at[sl], sem_p))
        for cp in cps:
            cp.wait()

        def tok_body(t, c0):
            es = [rows_v[t, pl.ds(16 * k, 16)] + prow_v[t, pl.ds(16 * k, 16)]
                  for k in range(8)]
            s = es[0]
            for k in range(1, 8):
                s = s + es[k]
            mu = _butterfly_sum(s, lane) * (1.0 / D)
            dvs = [e - mu for e in es]
            q = dvs[0] * dvs[0]
            for k in range(1, 8):
                q = q + dvs[k] * dvs[k]
            xv = _butterfly_sum(q, lane) * (1.0 / D) + EPS
            yi = jnp.int32(0x5F3759DF) - (lax.bitcast_convert_type(
                xv, jnp.int32) >> 1)
            y = lax.bitcast_convert_type(yi, jnp.float32)
            for _ in range(3):
                y = y * (1.5 - 0.5 * xv * y * y)
            for k in range(8):
                out_v[t, pl.ds(16 * k, 16)] = dvs[k] * y * gs[k] + bs[k]
            return c0
        lax.fori_loop(0, L, tok_body, 0, unroll=False)
        pltpu.sync_copy(out_v.at[pl.ds(0, L)], out_hbm.at[row])
        return carry0
    lax.fori_loop(0, ROWS_PER_W, row_body, 0, unroll=False)


@jax.jit
def kernel(input_ids, token_emb, pos_emb, gamma, beta):
    mesh = plsc.VectorSubcoreMesh(core_axis_name="c", subcore_axis_name="s")
    f = functools.partial(
        pl.kernel,
        mesh=mesh,
        out_type=jax.ShapeDtypeStruct((B, L, D), jnp.float32),
        scratch_types=[
            pltpu.VMEM((LP,), jnp.int32),
            pltpu.VMEM((LP,), jnp.int32),
            pltpu.VMEM((LP, D), jnp.float32),
            pltpu.VMEM((LP, D), jnp.float32),
            pltpu.VMEM((LP, D), jnp.float32),
            pltpu.VMEM((D,), jnp.float32),
            pltpu.VMEM((D,), jnp.float32),
            pltpu.SemaphoreType.DMA,
            pltpu.SemaphoreType.DMA,
        ],
    )(_sc_body)
    return f(input_ids.reshape(-1), token_emb, pos_emb, gamma, beta)


# speedup vs baseline: 5.5683x; 2.0655x over previous
import functools

import jax
import jax.numpy as jnp
from jax import lax
from jax.experimental import pallas as pl
from jax.experimental.pallas import tpu as pltpu
from jax.experimental.pallas import tpu_sc as plsc

D = 128
B = 1024
L = 200
EPS = 1e-12
LP = 208
NC = 2
NS = 16
NW = NC * NS
RPW = B // NW
NCHUNK = 13

_DNUMS = lax.GatherDimensionNumbers(
    offset_dims=(), collapsed_slice_dims=(0,), start_index_map=(0,))


def _take(v, idx):
    return lax.gather(v, idx[:, None], _DNUMS, (1,),
                      mode=lax.GatherScatterMode.PROMISE_IN_BOUNDS)


def _butterfly_sum(v, lane):
    for sh in (1, 2, 4, 8):
        v = v + _take(v, lane ^ sh)
    return v


def _sc_body(ids_hbm, tok_hbm, pos_hbm, gamma_hbm, beta_hbm, out_hbm,
             ids_v, pidx_v, rows0, rows1, prow0, prow1, gam_v, bet_v,
             sem_s, sem_t0, sem_t1, sem_p0, sem_p1, sem_o0, sem_o1):
    wid = lax.axis_index("s") * NC + lax.axis_index("c")
    pltpu.sync_copy(gamma_hbm, gam_v)
    pltpu.sync_copy(beta_hbm, bet_v)
    lane = lax.iota(jnp.int32, 16)
    gs = [gam_v[pl.ds(16 * k, 16)] for k in range(8)]
    bs = [bet_v[pl.ds(16 * k, 16)] for k in range(8)]
    rows = (rows0, rows1)
    prow = (prow0, prow1)
    sem_t = (sem_t0, sem_t1)
    sem_p = (sem_p0, sem_p1)
    sem_o = (sem_o0, sem_o1)
    base = wid * RPW

    stage = [pltpu.async_copy(ids_hbm.at[pl.ds((base + r) * L, L)],
                              ids_v.at[pl.ds(r * LP, L)], sem_s)
             for r in range(RPW)]
    for cp in stage:
        cp.wait()

    def pidx_row(r, c0):
        carry = lane * 0
        for k in range(NCHUNK):
            ids_k = ids_v[pl.ds(r * LP + 16 * k, 16)]
            m = jnp.minimum(ids_k, 1)
            c = m
            for sh in (1, 2, 4, 8):
                c = c + jnp.where(lane >= sh,
                                  _take(c, jnp.maximum(lane - sh, 0)), 0)
            pidx_v[pl.ds(r * LP + 16 * k, 16)] = (c + carry) * m
            carry = carry + _take(c, lane * 0 + 15)
        return c0
    lax.fori_loop(0, RPW, pidx_row, 0, unroll=False)

    def gathers(r, b, issue):
        cps = []
        for (off, n) in ((0, 104), (104, 96)):
            src_t = tok_hbm.at[ids_v.at[pl.ds(r * LP + off, n)]]
            src_p = pos_hbm.at[pidx_v.at[pl.ds(r * LP + off, n)]]
            dst_t = rows[b].at[pl.ds(off, n)]
            dst_p = prow[b].at[pl.ds(off, n)]
            if issue:
                cps.append(pltpu.async_copy(src_t, dst_t, sem_t[b]))
                cps.append(pltpu.async_copy(src_p, dst_p, sem_p[b]))
            else:
                cps.append(pltpu.make_async_copy(src_t, dst_t, sem_t[b]))
                cps.append(pltpu.make_async_copy(src_p, dst_p, sem_p[b]))
        return cps

    def wait_gathers(r, b):
        for cp in gathers(r, b, issue=False):
            cp.wait()

    def layernorm_row(b):
        rb, pb = rows[b], prow[b]

        def one(t):
            es = [rb[t, pl.ds(16 * k, 16)] + pb[t, pl.ds(16 * k, 16)]
                  for k in range(8)]
            s = es[0]
            q = es[0] * es[0]
            for k in range(1, 8):
                s = s + es[k]
                q = q + es[k] * es[k]
            mu = _butterfly_sum(s, lane) * (1.0 / D)
            ex2 = _butterfly_sum(q, lane) * (1.0 / D)
            xv = (ex2 - mu * mu) + EPS
            yi = jnp.int32(0x5F3759DF) - (lax.bitcast_convert_type(
                xv, jnp.int32) >> 1)
            y = lax.bitcast_convert_type(yi, jnp.float32)
            hx = 0.5 * xv
            for _ in range(3):
                y = y * (1.5 - hx * y * y)
            for k in range(8):
                gk = y * gs[k]
                rb[t, pl.ds(16 * k, 16)] = es[k] * gk - (mu * gk - bs[k])

        def pair(i, c0):
            one(2 * i)
            one(2 * i + 1)
            return c0
        lax.fori_loop(0, L // 2, pair, 0, unroll=False)

    def out_copy(r, b, issue):
        src = rows[b].at[pl.ds(0, L)]
        dst = out_hbm.at[base + r]
        if issue:
            return pltpu.async_copy(src, dst, sem_o[b])
        return pltpu.make_async_copy(src, dst, sem_o[b])

    def group(g, peel_first, peel_last):
        r0, r1 = 2 * g, 2 * g + 1
        if not peel_first:
            out_copy(r1 - 2, 1, issue=False).wait()
        gathers(r1, 1, issue=True)
        wait_gathers(r0, 0)
        layernorm_row(0)
        out_copy(r0, 0, issue=True)
        if not peel_last:
            out_copy(r0, 0, issue=False).wait()
            gathers(r0 + 2, 0, issue=True)
        wait_gathers(r1, 1)
        layernorm_row(1)
        out_copy(r1, 1, issue=True)

    gathers(0, 0, issue=True)
    group(0, peel_first=True, peel_last=False)

    def mid(g, c0):
        group(g, peel_first=False, peel_last=False)
        return c0
    lax.fori_loop(1, RPW // 2 - 1, mid, 0, unroll=False)

    group(RPW // 2 - 1, peel_first=False, peel_last=True)
    out_copy(RPW - 2, 0, issue=False).wait()
    out_copy(RPW - 1, 1, issue=False).wait()


@jax.jit
def kernel(input_ids, token_emb, pos_emb, gamma, beta):
    mesh = plsc.VectorSubcoreMesh(core_axis_name="c", subcore_axis_name="s")
    f = functools.partial(
        pl.kernel,
        mesh=mesh,
        out_type=jax.ShapeDtypeStruct((B, L, D), jnp.float32),
        scratch_types=[
            pltpu.VMEM((RPW * LP,), jnp.int32),
            pltpu.VMEM((RPW * LP,), jnp.int32),
            pltpu.VMEM((L, D), jnp.float32),
            pltpu.VMEM((L, D), jnp.float32),
            pltpu.VMEM((L, D), jnp.float32),
            pltpu.VMEM((L, D), jnp.float32),
            pltpu.VMEM((D,), jnp.float32),
            pltpu.VMEM((D,), jnp.float32),
            pltpu.SemaphoreType.DMA,
            pltpu.SemaphoreType.DMA,
            pltpu.SemaphoreType.DMA,
            pltpu.SemaphoreType.DMA,
            pltpu.SemaphoreType.DMA,
            pltpu.SemaphoreType.DMA,
            pltpu.SemaphoreType.DMA,
        ],
    )(_sc_body)
    return f(input_ids.reshape(-1), token_emb, pos_emb, gamma, beta)


# speedup vs baseline: 5.9797x; 1.0739x over previous
import functools

import jax
import jax.numpy as jnp
from jax import lax
from jax.experimental import pallas as pl
from jax.experimental.pallas import tpu as pltpu
from jax.experimental.pallas import tpu_sc as plsc

D = 128
B = 1024
L = 200
EPS = 1e-12
LP = 208
NC = 2
NS = 16
NW = NC * NS
RPW = B // NW
NCHUNK = 13
NBLK = 13

_DNUMS = lax.GatherDimensionNumbers(
    offset_dims=(), collapsed_slice_dims=(0,), start_index_map=(0,))


def _take(v, idx):
    return lax.gather(v, idx[:, None], _DNUMS, (1,),
                      mode=lax.GatherScatterMode.PROMISE_IN_BOUNDS)


def _tree_sum(vs):
    while len(vs) > 1:
        vs = [a + b for a, b in zip(vs[::2], vs[1::2])]
    return vs[0]


def _butterfly_sum(v, lane):
    for sh in (1, 2, 4, 8):
        v = v + _take(v, lane ^ sh)
    return v


def _sc_body(ids_hbm, tok_hbm, pos_hbm, gamma_hbm, beta_hbm, out_hbm,
             ids_v, pidx_v, rows0, rows1, prow0, prow1, gam_v, bet_v,
             sem_s, sem_t0, sem_t1, sem_p0, sem_p1, sem_o0, sem_o1):
    wid = lax.axis_index("s") * NC + lax.axis_index("c")
    pltpu.sync_copy(gamma_hbm, gam_v)
    pltpu.sync_copy(beta_hbm, bet_v)
    lane = lax.iota(jnp.int32, 16)
    lane16 = lane * 16
    gs = [gam_v[pl.ds(16 * k, 16)] for k in range(8)]
    bs = [bet_v[pl.ds(16 * k, 16)] for k in range(8)]
    rows = (rows0, rows1)
    prow = (prow0, prow1)
    sem_t = (sem_t0, sem_t1)
    sem_p = (sem_p0, sem_p1)
    sem_o = (sem_o0, sem_o1)
    base = wid * RPW

    stage = [pltpu.async_copy(ids_hbm.at[pl.ds((base + r) * L, L)],
                              ids_v.at[pl.ds(r * LP, L)], sem_s)
             for r in range(RPW)]
    for cp in stage:
        cp.wait()

    def pidx_row(r, c0):
        carry = lane * 0
        for k in range(NCHUNK):
            ids_k = ids_v[pl.ds(r * LP + 16 * k, 16)]
            m = jnp.minimum(ids_k, 1)
            c = m
            for sh in (1, 2, 4, 8):
                c = c + jnp.where(lane >= sh,
                                  _take(c, jnp.maximum(lane - sh, 0)), 0)
            pidx_v[pl.ds(r * LP + 16 * k, 16)] = (c + carry) * m
            carry = carry + _take(c, lane * 0 + 15)
        return c0
    lax.fori_loop(0, RPW, pidx_row, 0, unroll=False)

    def gathers(r, b, issue):
        cps = []
        for (off, n) in ((0, 104), (104, 96)):
            src_t = tok_hbm.at[ids_v.at[pl.ds(r * LP + off, n)]]
            src_p = pos_hbm.at[pidx_v.at[pl.ds(r * LP + off, n)]]
            dst_t = rows[b].at[pl.ds(off, n)]
            dst_p = prow[b].at[pl.ds(off, n)]
            if issue:
                cps.append(pltpu.async_copy(src_t, dst_t, sem_t[b]))
                cps.append(pltpu.async_copy(src_p, dst_p, sem_p[b]))
            else:
                cps.append(pltpu.make_async_copy(src_t, dst_t, sem_t[b]))
                cps.append(pltpu.make_async_copy(src_p, dst_p, sem_p[b]))
        return cps

    def wait_gathers(r, b):
        for cp in gathers(r, b, issue=False):
            cp.wait()

    def layernorm_row(b):
        rb, pb = rows[b], prow[b]

        def block(bi, c0):
            t0 = 16 * bi
            var_pack = lane * 0.0
            for j in range(16):
                t = t0 + j
                es = [rb[t, pl.ds(16 * k, 16)] + pb[t, pl.ds(16 * k, 16)]
                      for k in range(8)]
                mu = _butterfly_sum(_tree_sum(es), lane) * (1.0 / D)
                dv = [e - mu for e in es]
                for k in range(8):
                    rb[t, pl.ds(16 * k, 16)] = dv[k]
                var = _butterfly_sum(_tree_sum([d * d for d in dv]),
                                     lane) * (1.0 / D)
                var_pack = jnp.where(lane == j, var, var_pack)
            xv = jnp.maximum(var_pack, 0.0) + EPS
            yi = jnp.int32(0x5F3759DF) - (lax.bitcast_convert_type(
                xv, jnp.int32) >> 1)
            y = lax.bitcast_convert_type(yi, jnp.float32)
            hx = 0.5 * xv
            for _ in range(2):
                y = y * (1.5 - hx * y * y)
            for j in range(16):
                t = t0 + j
                yj = _take(y, jnp.full((16,), j, jnp.int32))
                for k in range(8):
                    dk = rb[t, pl.ds(16 * k, 16)]
                    rb[t, pl.ds(16 * k, 16)] = dk * (yj * gs[k]) + bs[k]
            return c0
        lax.fori_loop(0, NBLK, block, 0, unroll=False)

    def out_copy(r, b, issue):
        src = rows[b].at[pl.ds(0, L)]
        dst = out_hbm.at[base + r]
        if issue:
            return pltpu.async_copy(src, dst, sem_o[b])
        return pltpu.make_async_copy(src, dst, sem_o[b])

    def group(g, c0):
        r0, r1 = 2 * g, 2 * g + 1

        @pl.when(g > 0)
        def _():
            out_copy(r1 - 2, 1, issue=False).wait()
        gathers(r1, 1, issue=True)
        wait_gathers(r0, 0)
        layernorm_row(0)
        out_copy(r0, 0, issue=True)

        @pl.when(g < RPW // 2 - 1)
        def _():
            out_copy(r0, 0, issue=False).wait()
            gathers(r0 + 2, 0, issue=True)
        wait_gathers(r1, 1)
        layernorm_row(1)
        out_copy(r1, 1, issue=True)
        return c0

    gathers(0, 0, issue=True)
    lax.fori_loop(0, RPW // 2, group, 0, unroll=False)
    out_copy(RPW - 2, 0, issue=False).wait()
    out_copy(RPW - 1, 1, issue=False).wait()


@jax.jit
def kernel(input_ids, token_emb, pos_emb, gamma, beta):
    mesh = plsc.VectorSubcoreMesh(core_axis_name="c", subcore_axis_name="s")
    f = functools.partial(
        pl.kernel,
        mesh=mesh,
        out_type=jax.ShapeDtypeStruct((B, L, D), jnp.float32),
        scratch_types=[
            pltpu.VMEM((RPW * LP,), jnp.int32),
            pltpu.VMEM((RPW * LP,), jnp.int32),
            pltpu.VMEM((LP, D), jnp.float32),
            pltpu.VMEM((LP, D), jnp.float32),
            pltpu.VMEM((LP, D), jnp.float32),
            pltpu.VMEM((LP, D), jnp.float32),
            pltpu.VMEM((D,), jnp.float32),
            pltpu.VMEM((D,), jnp.float32),
            pltpu.SemaphoreType.DMA,
            pltpu.SemaphoreType.DMA,
            pltpu.SemaphoreType.DMA,
            pltpu.SemaphoreType.DMA,
            pltpu.SemaphoreType.DMA,
            pltpu.SemaphoreType.DMA,
            pltpu.SemaphoreType.DMA,
        ],
    )(_sc_body)
    return f(input_ids.reshape(-1), token_emb, pos_emb, gamma, beta)


# speedup vs baseline: 7.5520x; 1.2629x over previous
import functools

import jax
import jax.numpy as jnp
from jax import lax
from jax.experimental import pallas as pl
from jax.experimental.pallas import tpu as pltpu
from jax.experimental.pallas import tpu_sc as plsc

D = 128
B = 1024
L = 200
EPS = 1e-12
LP = 208
NC = 2
NS = 16
NW = NC * NS
RPW = B // NW
NCHUNK = 13
NBLK = 13

_DNUMS = lax.GatherDimensionNumbers(
    offset_dims=(), collapsed_slice_dims=(0,), start_index_map=(0,))


def _take(v, idx):
    return lax.gather(v, idx[:, None], _DNUMS, (1,),
                      mode=lax.GatherScatterMode.PROMISE_IN_BOUNDS)


def _tree_sum(vs):
    while len(vs) > 1:
        vs = [a + b for a, b in zip(vs[::2], vs[1::2])]
    return vs[0]


def _butterfly_sum(v, lane):
    for sh in (1, 2, 4, 8):
        v = v + _take(v, lane ^ sh)
    return v


def _sc_body(ids_hbm, tok_hbm, pos_hbm, gamma_hbm, beta_hbm, out_hbm,
             ids_v, pidx_v, rows0, rows1, prow0, prow1, gam_v, bet_v,
             shared_pos, sem_s, sem_t0, sem_t1, sem_p0, sem_p1, sem_o0, sem_o1):
    wid = lax.axis_index("s") * NC + lax.axis_index("c")
    pltpu.sync_copy(gamma_hbm, gam_v)
    pltpu.sync_copy(beta_hbm, bet_v)
    lane = lax.iota(jnp.int32, 16)
    lane16 = lane * 16
    gs = [gam_v[pl.ds(16 * k, 16)] for k in range(8)]
    bs = [bet_v[pl.ds(16 * k, 16)] for k in range(8)]
    rows = (rows0, rows1)
    prow = (prow0, prow1)
    sem_t = (sem_t0, sem_t1)
    sem_p = (sem_p0, sem_p1)
    sem_o = (sem_o0, sem_o1)
    base = wid * RPW

    @pl.when(lax.axis_index("s") == 0)
    def _():
        pltpu.sync_copy(pos_hbm.at[pl.ds(0, LP)], shared_pos)
    plsc.subcore_barrier()

    stage = [pltpu.async_copy(ids_hbm.at[pl.ds((base + r) * L, L)],
                              ids_v.at[pl.ds(r * LP, L)], sem_s)
             for r in range(RPW)]
    for cp in stage:
        cp.wait()

    def pidx_row(r, c0):
        carry = lane * 0
        for k in range(NCHUNK):
            ids_k = ids_v[pl.ds(r * LP + 16 * k, 16)]
            m = jnp.minimum(ids_k, 1)
            c = m
            for sh in (1, 2, 4, 8):
                c = c + jnp.where(lane >= sh,
                                  _take(c, jnp.maximum(lane - sh, 0)), 0)
            pidx_v[pl.ds(r * LP + 16 * k, 16)] = (c + carry) * m
            carry = carry + _take(c, lane * 0 + 15)
        return c0
    lax.fori_loop(0, RPW, pidx_row, 0, unroll=False)

    def gathers(r, b, issue):
        cps = []
        for (off, n) in ((0, 104), (104, 96)):
            src_t = tok_hbm.at[ids_v.at[pl.ds(r * LP + off, n)]]
            src_p = shared_pos.at[pidx_v.at[pl.ds(r * LP + off, n)]]
            dst_t = rows[b].at[pl.ds(off, n)]
            dst_p = prow[b].at[pl.ds(off, n)]
            if issue:
                cps.append(pltpu.async_copy(src_t, dst_t, sem_t[b]))
                cps.append(pltpu.async_copy(src_p, dst_p, sem_p[b]))
            else:
                cps.append(pltpu.make_async_copy(src_t, dst_t, sem_t[b]))
                cps.append(pltpu.make_async_copy(src_p, dst_p, sem_p[b]))
        return cps

    def wait_gathers(r, b):
        for cp in gathers(r, b, issue=False):
            cp.wait()

    def layernorm_row(b):
        rb, pb = rows[b], prow[b]

        def block(bi, c0):
            t0 = 16 * bi
            var_pack = lane * 0.0
            for j in range(16):
                t = t0 + j
                es = [rb[t, pl.ds(16 * k, 16)] + pb[t, pl.ds(16 * k, 16)]
                      for k in range(8)]
                mu = _butterfly_sum(_tree_sum(es), lane) * (1.0 / D)
                dv = [e - mu for e in es]
                for k in range(8):
                    rb[t, pl.ds(16 * k, 16)] = dv[k]
                var = _butterfly_sum(_tree_sum([d * d for d in dv]),
                                     lane) * (1.0 / D)
                var_pack = jnp.where(lane == j, var, var_pack)
            xv = jnp.maximum(var_pack, 0.0) + EPS
            yi = jnp.int32(0x5F3759DF) - (lax.bitcast_convert_type(
                xv, jnp.int32) >> 1)
            y = lax.bitcast_convert_type(yi, jnp.float32)
            hx = 0.5 * xv
            for _ in range(2):
                y = y * (1.5 - hx * y * y)
            for j in range(16):
                t = t0 + j
                yj = _take(y, jnp.full((16,), j, jnp.int32))
                for k in range(8):
                    dk = rb[t, pl.ds(16 * k, 16)]
                    rb[t, pl.ds(16 * k, 16)] = dk * (yj * gs[k]) + bs[k]
            return c0
        lax.fori_loop(0, NBLK, block, 0, unroll=False)

    def out_copy(r, b, issue):
        src = rows[b].at[pl.ds(0, L)]
        dst = out_hbm.at[base + r]
        if issue:
            return pltpu.async_copy(src, dst, sem_o[b])
        return pltpu.make_async_copy(src, dst, sem_o[b])

    def group(g, c0):
        r0, r1 = 2 * g, 2 * g + 1

        @pl.when(g > 0)
        def _():
            out_copy(r1 - 2, 1, issue=False).wait()
        gathers(r1, 1, issue=True)
        wait_gathers(r0, 0)
        layernorm_row(0)
        out_copy(r0, 0, issue=True)

        @pl.when(g < RPW // 2 - 1)
        def _():
            out_copy(r0, 0, issue=False).wait()
            gathers(r0 + 2, 0, issue=True)
        wait_gathers(r1, 1)
        layernorm_row(1)
        out_copy(r1, 1, issue=True)
        return c0

    gathers(0, 0, issue=True)
    lax.fori_loop(0, RPW // 2, group, 0, unroll=False)
    out_copy(RPW - 2, 0, issue=False).wait()
    out_copy(RPW - 1, 1, issue=False).wait()


@jax.jit
def kernel(input_ids, token_emb, pos_emb, gamma, beta):
    mesh = plsc.VectorSubcoreMesh(core_axis_name="c", subcore_axis_name="s")
    f = functools.partial(
        pl.kernel,
        mesh=mesh,
        out_type=jax.ShapeDtypeStruct((B, L, D), jnp.float32),
        scratch_types=[
            pltpu.VMEM((RPW * LP,), jnp.int32),
            pltpu.VMEM((RPW * LP,), jnp.int32),
            pltpu.VMEM((LP, D), jnp.float32),
            pltpu.VMEM((LP, D), jnp.float32),
            pltpu.VMEM((LP, D), jnp.float32),
            pltpu.VMEM((LP, D), jnp.float32),
            pltpu.VMEM((D,), jnp.float32),
            pltpu.VMEM((D,), jnp.float32),
            pltpu.VMEM_SHARED((208, D), jnp.float32),
            pltpu.SemaphoreType.DMA,
            pltpu.SemaphoreType.DMA,
            pltpu.SemaphoreType.DMA,
            pltpu.SemaphoreType.DMA,
            pltpu.SemaphoreType.DMA,
            pltpu.SemaphoreType.DMA,
            pltpu.SemaphoreType.DMA,
        ],
    )(_sc_body)
    return f(input_ids.reshape(-1), token_emb, pos_emb, gamma, beta)
